# baseline (device time: 43888 ns/iter reference)
import os

import numpy as np
import jax
import jax.numpy as jnp
from jax import lax
from jax.experimental import pallas as pl
from jax.experimental.pallas import tpu as pltpu

N_DEV = 8
B, SQ, D = 2, 256, 768
HL, DH = 4, 64
HD = HL * DH


def _consts():
    inv = 1.0 / (10000.0 ** (np.arange(0, DH, 2) / DH))
    pos = np.arange(SQ)[:, None] * inv[None, :]
    cos = np.repeat(np.cos(pos), 2, axis=-1)
    sin = np.repeat(np.sin(pos), 2, axis=-1)
    cosp = np.tile(cos, (1, HL)).astype(np.float32)
    sinp = np.tile(sin, (1, HL)).astype(np.float32)
    P = np.zeros((DH, DH), np.float32)
    for k in range(DH // 2):
        P[2 * k + 1, 2 * k] = -1.0
        P[2 * k, 2 * k + 1] = 1.0
    Pp = np.kron(np.eye(HL, dtype=np.float32), P)
    return cosp, sinp, Pp


BITS_RS = (0, 2, 1)
RS_ROWS = (128, 64, 32)
REGS = ((0, 128), (128, 64), (192, 32),
        (224, 32), (256, 64), (320, 128))


def _bit(v, k):
    return jnp.bitwise_and(lax.shift_right_logical(v, k), 1)


def _body(x_ref, wq_ref, wk_ref, wv_ref, wo_ref, cos_ref, sin_ref, rot_ref,
          out_ref, send_ref, recv_ref, q_s, k_s, v_s, send_sems, recv_sems):
    my = lax.axis_index("i")

    barrier_sem = pltpu.get_barrier_semaphore()
    for b in (0, 1, 2):
        pl.semaphore_signal(barrier_sem, inc=1,
                            device_id=(jnp.bitwise_xor(my, 1 << b),),
                            device_id_type=pl.DeviceIdType.MESH)
    pl.semaphore_wait(barrier_sem, 3)

    cos = cos_ref[:, :]
    sin = sin_ref[:, :]
    rot = rot_ref[:, :].astype(jnp.bfloat16)
    bf = jnp.bfloat16
    f32 = jnp.float32
    compute_only = os.environ.get("KERNEL_COMPUTE_ONLY") == "1"

    wq = wq_ref[:, :].astype(bf)
    wk = wk_ref[:, :].astype(bf)
    wv = wv_ref[:, :].astype(bf)
    wo = wo_ref[:, :].astype(bf)
    for b in range(B):
        xb = x_ref[b, :, :].astype(bf)
        q = jnp.dot(xb, wq, preferred_element_type=f32)
        k = jnp.dot(xb, wk, preferred_element_type=f32)
        v = jnp.dot(xb, wv, preferred_element_type=f32)
        q_s[b, :, :] = (
            q * cos + jnp.dot(q.astype(bf), rot, preferred_element_type=f32) * sin)
        k_s[b, :, :] = (
            k * cos + jnp.dot(k.astype(bf), rot, preferred_element_type=f32) * sin
        ).astype(bf)
        v_s[b, :, :] = v.astype(bf)

    colr = lax.broadcasted_iota(jnp.int32, (SQ // 2, HD), 1)

    def attn_rows(r0, send_off):
        for b in range(B):
            qr = q_s[b, pl.ds(r0, SQ // 2), :]
            kf = k_s[b, :, :]
            vf = v_s[b, :, :]
            ctx = jnp.zeros((SQ // 2, HD), f32)
            for h in range(HL):
                m = (colr >= h * DH) & (colr < (h + 1) * DH)
                qm = jnp.where(m, qr, 0.0).astype(bf)
                s = lax.dot_general(qm, kf, (((1,), (1,)), ((), ())),
                                    preferred_element_type=f32) * 0.125
                w = jnp.exp(s)
                w = (w / jnp.sum(w, axis=-1, keepdims=True)).astype(bf)
                u = jnp.dot(w, vf, preferred_element_type=f32)
                ctx = ctx + jnp.where(m, u, 0.0)
            part = jnp.dot(ctx.astype(bf), wo, preferred_element_type=f32)
            out_ref[b, pl.ds(r0, SQ // 2), :] = part
            if send_off is not None:
                send_ref[b, send_off:send_off + SQ // 2, :] = part.astype(bf)

    half = SQ // 2
    mb = _bit(my, BITS_RS[0])
    partner = jnp.bitwise_xor(my, 1 << BITS_RS[0])
    keep_lo = pl.multiple_of(mb * half, 32)
    send_lo = pl.multiple_of((1 - mb) * half, 32)
    off, rows = REGS[0]
    attn_rows(send_lo, off)
    rdma = pltpu.make_async_remote_copy(
        src_ref=send_ref.at[:, pl.ds(off, rows), :],
        dst_ref=recv_ref.at[:, pl.ds(off, rows), :],
        send_sem=send_sems.at[0],
        recv_sem=recv_sems.at[0],
        device_id=(partner,),
        device_id_type=pl.DeviceIdType.MESH,
    )
    if not compute_only:
        rdma.start()
    attn_rows(keep_lo, None)
    if compute_only:
        return
    rdma.wait()
    out_ref[:, pl.ds(keep_lo, half), :] = (
        out_ref[:, pl.ds(keep_lo, half), :]
        + recv_ref[:, off:off + rows, :].astype(f32))
    lo = keep_lo
    sz = half

    for r, bpos in list(enumerate(BITS_RS))[1:]:
        half = sz // 2
        mb = _bit(my, bpos)
        partner = jnp.bitwise_xor(my, 1 << bpos)
        keep_lo = pl.multiple_of(lo + mb * half, 32)
        send_lo = pl.multiple_of(lo + (1 - mb) * half, 32)
        off, rows = REGS[r]
        send_ref[:, off:off + rows, :] = (
            out_ref[:, pl.ds(send_lo, half), :].astype(jnp.bfloat16))
        rdma = pltpu.make_async_remote_copy(
            src_ref=send_ref.at[:, pl.ds(off, rows), :],
            dst_ref=recv_ref.at[:, pl.ds(off, rows), :],
            send_sem=send_sems.at[r],
            recv_sem=recv_sems.at[r],
            device_id=(partner,),
            device_id_type=pl.DeviceIdType.MESH,
        )
        rdma.start()
        rdma.wait()
        out_ref[:, pl.ds(keep_lo, half), :] = (
            out_ref[:, pl.ds(keep_lo, half), :]
            + recv_ref[:, off:off + rows, :].astype(jnp.float32))
        lo = keep_lo
        sz = half

    for j, bpos in enumerate(reversed(BITS_RS)):
        mb = _bit(my, bpos)
        partner = jnp.bitwise_xor(my, 1 << bpos)
        plo = pl.multiple_of(lo + sz - 2 * mb * sz, 32)
        lo = pl.multiple_of(lo, 32)
        off, rows = REGS[3 + j]
        send_ref[:, off:off + rows, :] = (
            out_ref[:, pl.ds(lo, sz), :].astype(jnp.bfloat16))
        rdma = pltpu.make_async_remote_copy(
            src_ref=send_ref.at[:, pl.ds(off, rows), :],
            dst_ref=recv_ref.at[:, pl.ds(off, rows), :],
            send_sem=send_sems.at[3 + j],
            recv_sem=recv_sems.at[3 + j],
            device_id=(partner,),
            device_id_type=pl.DeviceIdType.MESH,
        )
        rdma.start()
        rdma.wait()
        out_ref[:, pl.ds(plo, sz), :] = (
            recv_ref[:, off:off + rows, :].astype(jnp.float32))
        lo = jnp.minimum(lo, plo)
        sz = sz * 2


def kernel(x, Wq, Wk, Wv, Wo):
    cosp, sinp, Pp = _consts()
    return pl.pallas_call(
        _body,
        out_shape=jax.ShapeDtypeStruct((B, SQ, D), jnp.float32),
        in_specs=[pl.BlockSpec(memory_space=pltpu.VMEM)] * 8,
        out_specs=pl.BlockSpec(memory_space=pltpu.VMEM),
        scratch_shapes=[
            pltpu.VMEM((B, 448, D), jnp.bfloat16),
            pltpu.VMEM((B, 448, D), jnp.bfloat16),
            pltpu.VMEM((B, SQ, HD), jnp.float32),
            pltpu.VMEM((B, SQ, HD), jnp.bfloat16),
            pltpu.VMEM((B, SQ, HD), jnp.bfloat16),
            pltpu.SemaphoreType.DMA((6,)),
            pltpu.SemaphoreType.DMA((6,)),
        ],
        compiler_params=pltpu.CompilerParams(collective_id=0),
    )(x, Wq, Wk, Wv, Wo, jnp.asarray(cosp), jnp.asarray(sinp), jnp.asarray(Pp))


# device time: 35188 ns/iter; 1.2472x vs baseline; 1.2472x over previous
import os

import numpy as np
import jax
import jax.numpy as jnp
from jax import lax
from jax.experimental import pallas as pl
from jax.experimental.pallas import tpu as pltpu

N_DEV = 8
B, SQ, D = 2, 256, 768
HL, DH = 4, 64
HD = HL * DH
SEG = SQ // N_DEV


def _consts():
    inv = 1.0 / (10000.0 ** (np.arange(0, DH, 2) / DH))
    pos = np.arange(SQ)[:, None] * inv[None, :]
    cos = np.repeat(np.cos(pos), 2, axis=-1)
    sin = np.repeat(np.sin(pos), 2, axis=-1)
    cosp = np.tile(cos, (1, HL)).astype(np.float32)
    sinp = np.tile(sin, (1, HL)).astype(np.float32)
    P = np.zeros((DH, DH), np.float32)
    for k in range(DH // 2):
        P[2 * k + 1, 2 * k] = -1.0
        P[2 * k, 2 * k + 1] = 1.0
    Pp = np.kron(np.eye(HL, dtype=np.float32), P)
    return cosp, sinp, Pp


def _body(x_ref, wq_ref, wk_ref, wv_ref, wo_ref, cos_ref, sin_ref, rot_ref,
          out_ref, rs_send, rs_recv, ag_send, ag_recv,
          rs_ssem, rs_rsem, ag_ssem, ag_rsem):
    my = lax.axis_index("i")
    bf = jnp.bfloat16
    f32 = jnp.float32
    compute_only = os.environ.get("KERNEL_COMPUTE_ONLY") == "1"

    barrier_sem = pltpu.get_barrier_semaphore()
    for o in range(1, N_DEV):
        pl.semaphore_signal(barrier_sem, inc=1,
                            device_id=(lax.rem(my + o, N_DEV),),
                            device_id_type=pl.DeviceIdType.MESH)
    pl.semaphore_wait(barrier_sem, N_DEV - 1)

    cos = cos_ref[:, :]
    sin = sin_ref[:, :]
    rot = rot_ref[:, :].astype(bf)

    wq = wq_ref[:, :].astype(bf)
    wk = wk_ref[:, :].astype(bf)
    wv = wv_ref[:, :].astype(bf)
    wo = wo_ref[:, :].astype(bf)
    qs, ks, vs = [], [], []
    for b in range(B):
        xb = x_ref[b, :, :].astype(bf)
        q = jnp.dot(xb, wq, preferred_element_type=f32)
        k = jnp.dot(xb, wk, preferred_element_type=f32)
        v = jnp.dot(xb, wv, preferred_element_type=f32)
        qs.append(q * cos + jnp.dot(q.astype(bf), rot,
                                    preferred_element_type=f32) * sin)
        ks.append((k * cos + jnp.dot(k.astype(bf), rot,
                                     preferred_element_type=f32) * sin).astype(bf))
        vs.append(v.astype(bf))

    half = SQ // 2
    colr = lax.broadcasted_iota(jnp.int32, (half, HD), 1)

    def attn_half(r0):
        for b in range(B):
            qr = qs[b][r0:r0 + half, :]
            kf, vf = ks[b], vs[b]
            ctx = jnp.zeros((half, HD), f32)
            for h in range(HL):
                m = (colr >= h * DH) & (colr < (h + 1) * DH)
                qm = jnp.where(m, qr, 0.0).astype(bf)
                s = lax.dot_general(qm, kf, (((1,), (1,)), ((), ())),
                                    preferred_element_type=f32) * 0.125
                w = jnp.exp(s)
                w = (w / jnp.sum(w, axis=-1, keepdims=True)).astype(bf)
                u = jnp.dot(w, vf, preferred_element_type=f32)
                ctx = ctx + jnp.where(m, u, 0.0)
            out_ref[b, r0:r0 + half, :] = jnp.dot(
                ctx.astype(bf), wo, preferred_element_type=f32)

    def scatter_sends(lo_half):
        for o in range(1, N_DEV):
            p = lax.rem(my + o, N_DEV)
            in_this_half = (p >= lo_half // SEG) & (p < (lo_half + half) // SEG)

            @pl.when(in_this_half)
            def _():
                src_lo = pl.multiple_of(p * SEG, SEG)
                rs_send[:, (o - 1) * SEG:o * SEG, :] = (
                    out_ref[:, pl.ds(src_lo, SEG), :].astype(bf))
                pltpu.make_async_remote_copy(
                    src_ref=rs_send.at[:, pl.ds((o - 1) * SEG, SEG), :],
                    dst_ref=rs_recv.at[:, pl.ds((o - 1) * SEG, SEG), :],
                    send_sem=rs_ssem.at[o - 1],
                    recv_sem=rs_rsem.at[o - 1],
                    device_id=(p,),
                    device_id_type=pl.DeviceIdType.MESH,
                ).start()

    attn_half(0)
    if not compute_only:
        scatter_sends(0)
    attn_half(half)
    if compute_only:
        return
    scatter_sends(half)

    for o in range(1, N_DEV):
        pltpu.make_async_remote_copy(
            src_ref=rs_send.at[:, pl.ds((o - 1) * SEG, SEG), :],
            dst_ref=rs_recv.at[:, pl.ds((o - 1) * SEG, SEG), :],
            send_sem=rs_ssem.at[o - 1],
            recv_sem=rs_rsem.at[o - 1],
            device_id=(my,),
            device_id_type=pl.DeviceIdType.MESH,
        ).wait_recv()
    my_lo = pl.multiple_of(my * SEG, SEG)
    acc = out_ref[:, pl.ds(my_lo, SEG), :]
    for o in range(1, N_DEV):
        acc = acc + rs_recv[:, (o - 1) * SEG:o * SEG, :].astype(f32)
    out_ref[:, pl.ds(my_lo, SEG), :] = acc

    ag_send[:, :, :] = acc.astype(bf)
    for o in range(1, N_DEV):
        p = lax.rem(my + o, N_DEV)
        pltpu.make_async_remote_copy(
            src_ref=ag_send,
            dst_ref=ag_recv.at[:, pl.ds((o - 1) * SEG, SEG), :],
            send_sem=ag_ssem.at[o - 1],
            recv_sem=ag_rsem.at[o - 1],
            device_id=(p,),
            device_id_type=pl.DeviceIdType.MESH,
        ).start()
    for o in range(1, N_DEV):
        rdma = pltpu.make_async_remote_copy(
            src_ref=ag_send,
            dst_ref=ag_recv.at[:, pl.ds((o - 1) * SEG, SEG), :],
            send_sem=ag_ssem.at[o - 1],
            recv_sem=ag_rsem.at[o - 1],
            device_id=(my,),
            device_id_type=pl.DeviceIdType.MESH,
        )
        rdma.wait_recv()
        s_o = pl.multiple_of(lax.rem(my - o + N_DEV, N_DEV) * SEG, SEG)
        out_ref[:, pl.ds(s_o, SEG), :] = (
            ag_recv[:, (o - 1) * SEG:o * SEG, :].astype(f32))
    for o in range(1, N_DEV):
        pltpu.make_async_remote_copy(
            src_ref=rs_send.at[:, pl.ds((o - 1) * SEG, SEG), :],
            dst_ref=rs_recv.at[:, pl.ds((o - 1) * SEG, SEG), :],
            send_sem=rs_ssem.at[o - 1],
            recv_sem=rs_rsem.at[o - 1],
            device_id=(my,),
            device_id_type=pl.DeviceIdType.MESH,
        ).wait_send()
        pltpu.make_async_remote_copy(
            src_ref=ag_send,
            dst_ref=ag_recv.at[:, pl.ds((o - 1) * SEG, SEG), :],
            send_sem=ag_ssem.at[o - 1],
            recv_sem=ag_rsem.at[o - 1],
            device_id=(my,),
            device_id_type=pl.DeviceIdType.MESH,
        ).wait_send()


def kernel(x, Wq, Wk, Wv, Wo):
    cosp, sinp, Pp = _consts()
    n = N_DEV - 1
    return pl.pallas_call(
        _body,
        out_shape=jax.ShapeDtypeStruct((B, SQ, D), jnp.float32),
        in_specs=[pl.BlockSpec(memory_space=pltpu.VMEM)] * 8,
        out_specs=pl.BlockSpec(memory_space=pltpu.VMEM),
        scratch_shapes=[
            pltpu.VMEM((B, n * SEG, D), jnp.bfloat16),
            pltpu.VMEM((B, n * SEG, D), jnp.bfloat16),
            pltpu.VMEM((B, SEG, D), jnp.bfloat16),
            pltpu.VMEM((B, n * SEG, D), jnp.bfloat16),
            pltpu.SemaphoreType.DMA((n,)),
            pltpu.SemaphoreType.DMA((n,)),
            pltpu.SemaphoreType.DMA((n,)),
            pltpu.SemaphoreType.DMA((n,)),
        ],
        compiler_params=pltpu.CompilerParams(collective_id=0),
    )(x, Wq, Wk, Wv, Wo, jnp.asarray(cosp), jnp.asarray(sinp), jnp.asarray(Pp))


# device time: 26562 ns/iter; 1.6523x vs baseline; 1.3247x over previous
import os

import numpy as np
import jax
import jax.numpy as jnp
from jax import lax
from jax.experimental import pallas as pl
from jax.experimental.pallas import tpu as pltpu

N_DEV = 8
B, SQ, D = 2, 256, 768
HL, DH = 4, 64
HD = HL * DH
SEG = SQ // N_DEV


def _consts():
    inv = 1.0 / (10000.0 ** (np.arange(0, DH, 2) / DH))
    pos = np.arange(SQ)[:, None] * inv[None, :]
    cos = np.repeat(np.cos(pos), 2, axis=-1)
    sin = np.repeat(np.sin(pos), 2, axis=-1)
    cosp = np.tile(cos, (1, HL)).astype(np.float32)
    sinp = np.tile(sin, (1, HL)).astype(np.float32)
    P = np.zeros((DH, DH), np.float32)
    for k in range(DH // 2):
        P[2 * k + 1, 2 * k] = -1.0
        P[2 * k, 2 * k + 1] = 1.0
    Pp = np.kron(np.eye(HL, dtype=np.float32), P)
    return cosp, sinp, Pp


def _body(x_ref, wq_ref, wk_ref, wv_ref, wo_ref, cos_ref, sin_ref, rot_ref,
          out_ref, q_sc, rs_send, rs_recv, ag_send, ag_recv,
          rs_ssem, rs_rsem, ag_ssem, ag_rsem):
    my = lax.axis_index("i")
    bf = jnp.bfloat16
    f32 = jnp.float32
    compute_only = os.environ.get("KERNEL_COMPUTE_ONLY") == "1"

    barrier_sem = pltpu.get_barrier_semaphore()
    for o in range(1, N_DEV):
        pl.semaphore_signal(barrier_sem, inc=1,
                            device_id=(lax.rem(my + o, N_DEV),),
                            device_id_type=pl.DeviceIdType.MESH)
    pl.semaphore_wait(barrier_sem, N_DEV - 1)

    cos = cos_ref[:, :]
    sin = sin_ref[:, :]
    rot = rot_ref[:, :].astype(bf)

    wq = wq_ref[:, :].astype(bf)
    wk = wk_ref[:, :].astype(bf)
    wv = wv_ref[:, :].astype(bf)
    wo = wo_ref[:, :].astype(bf)
    ks, vs = [], []
    for b in range(B):
        xb = x_ref[b, :, :].astype(bf)
        q = jnp.dot(xb, wq, preferred_element_type=f32)
        k = jnp.dot(xb, wk, preferred_element_type=f32)
        v = jnp.dot(xb, wv, preferred_element_type=f32)
        q_sc[b, :, :] = (q * cos + jnp.dot(q.astype(bf), rot,
                                           preferred_element_type=f32)
                         * sin).astype(bf)
        ks.append((k * cos + jnp.dot(k.astype(bf), rot,
                                     preferred_element_type=f32) * sin).astype(bf))
        vs.append(v.astype(bf))

    def attn_block(r0, nrows):
        colr = lax.broadcasted_iota(jnp.int32, (nrows, HD), 1)
        for b in range(B):
            qr = q_sc[b, pl.ds(r0, nrows), :]
            kf, vf = ks[b], vs[b]
            ctx = jnp.zeros((nrows, HD), f32)
            for h in range(HL):
                m = (colr >= h * DH) & (colr < (h + 1) * DH)
                qm = jnp.where(m, qr, 0).astype(bf)
                s = lax.dot_general(qm, kf, (((1,), (1,)), ((), ())),
                                    preferred_element_type=f32) * 0.125
                w = jnp.exp(s)
                w = (w / jnp.sum(w, axis=-1, keepdims=True)).astype(bf)
                u = jnp.dot(w, vf, preferred_element_type=f32)
                ctx = ctx + jnp.where(m, u, 0.0)
            out_ref[b, pl.ds(r0, nrows), :] = jnp.dot(
                ctx.astype(bf), wo, preferred_element_type=f32)

    def send_to(p):
        o = lax.rem(p - my + N_DEV, N_DEV)
        off = pl.multiple_of((o - 1) * SEG, SEG)
        src_lo = pl.multiple_of(p * SEG, SEG)
        rs_send[:, pl.ds(off, SEG), :] = (
            out_ref[:, pl.ds(src_lo, SEG), :].astype(bf))
        pltpu.make_async_remote_copy(
            src_ref=rs_send.at[:, pl.ds(off, SEG), :],
            dst_ref=rs_recv.at[:, pl.ds(off, SEG), :],
            send_sem=rs_ssem.at[o - 1],
            recv_sem=rs_rsem.at[o - 1],
            device_id=(p,),
            device_id_type=pl.DeviceIdType.MESH,
        ).start()

    if compute_only:
        for q in range(4):
            attn_block(pl.multiple_of(jnp.int32(q) * (2 * SEG), 2 * SEG),
                       2 * SEG)
        return

    q_my = my // 2
    for t in range(3):
        qq = lax.rem(q_my + 1 + t, 4)
        r0 = pl.multiple_of(qq * (2 * SEG), 2 * SEG)
        attn_block(r0, 2 * SEG)
        send_to(2 * qq)
        send_to(2 * qq + 1)
    partner = jnp.bitwise_xor(my, 1)
    attn_block(pl.multiple_of(partner * SEG, SEG), SEG)
    send_to(partner)
    attn_block(pl.multiple_of(my * SEG, SEG), SEG)

    my_lo = pl.multiple_of(my * SEG, SEG)
    acc = out_ref[:, pl.ds(my_lo, SEG), :]
    for o in (2, 6, 3, 5, 4, 1, 7):
        pltpu.make_async_remote_copy(
            src_ref=rs_send.at[:, pl.ds((o - 1) * SEG, SEG), :],
            dst_ref=rs_recv.at[:, pl.ds((o - 1) * SEG, SEG), :],
            send_sem=rs_ssem.at[o - 1],
            recv_sem=rs_rsem.at[o - 1],
            device_id=(my,),
            device_id_type=pl.DeviceIdType.MESH,
        ).wait_recv()
        acc = acc + rs_recv[:, (o - 1) * SEG:o * SEG, :].astype(f32)
    out_ref[:, pl.ds(my_lo, SEG), :] = acc

    ag_send[:, :, :] = acc.astype(bf)
    for o in (4, 3, 5, 2, 6, 1, 7):
        p = lax.rem(my + o, N_DEV)
        pltpu.make_async_remote_copy(
            src_ref=ag_send,
            dst_ref=ag_recv.at[:, pl.ds((o - 1) * SEG, SEG), :],
            send_sem=ag_ssem.at[o - 1],
            recv_sem=ag_rsem.at[o - 1],
            device_id=(p,),
            device_id_type=pl.DeviceIdType.MESH,
        ).start()
    for o in (1, 7, 2, 6, 3, 5, 4):
        rdma = pltpu.make_async_remote_copy(
            src_ref=ag_send,
            dst_ref=ag_recv.at[:, pl.ds((o - 1) * SEG, SEG), :],
            send_sem=ag_ssem.at[o - 1],
            recv_sem=ag_rsem.at[o - 1],
            device_id=(my,),
            device_id_type=pl.DeviceIdType.MESH,
        )
        rdma.wait_recv()
        s_o = pl.multiple_of(lax.rem(my - o + N_DEV, N_DEV) * SEG, SEG)
        out_ref[:, pl.ds(s_o, SEG), :] = (
            ag_recv[:, (o - 1) * SEG:o * SEG, :].astype(f32))
    for o in range(1, N_DEV):
        pltpu.make_async_remote_copy(
            src_ref=rs_send.at[:, pl.ds((o - 1) * SEG, SEG), :],
            dst_ref=rs_recv.at[:, pl.ds((o - 1) * SEG, SEG), :],
            send_sem=rs_ssem.at[o - 1],
            recv_sem=rs_rsem.at[o - 1],
            device_id=(my,),
            device_id_type=pl.DeviceIdType.MESH,
        ).wait_send()
        pltpu.make_async_remote_copy(
            src_ref=ag_send,
            dst_ref=ag_recv.at[:, pl.ds((o - 1) * SEG, SEG), :],
            send_sem=ag_ssem.at[o - 1],
            recv_sem=ag_rsem.at[o - 1],
            device_id=(my,),
            device_id_type=pl.DeviceIdType.MESH,
        ).wait_send()


def kernel(x, Wq, Wk, Wv, Wo):
    cosp, sinp, Pp = _consts()
    n = N_DEV - 1
    return pl.pallas_call(
        _body,
        out_shape=jax.ShapeDtypeStruct((B, SQ, D), jnp.float32),
        in_specs=[pl.BlockSpec(memory_space=pltpu.VMEM)] * 8,
        out_specs=pl.BlockSpec(memory_space=pltpu.VMEM),
        scratch_shapes=[
            pltpu.VMEM((B, SQ, HD), jnp.bfloat16),
            pltpu.VMEM((B, n * SEG, D), jnp.bfloat16),
            pltpu.VMEM((B, n * SEG, D), jnp.bfloat16),
            pltpu.VMEM((B, SEG, D), jnp.bfloat16),
            pltpu.VMEM((B, n * SEG, D), jnp.bfloat16),
            pltpu.SemaphoreType.DMA((n,)),
            pltpu.SemaphoreType.DMA((n,)),
            pltpu.SemaphoreType.DMA((n,)),
            pltpu.SemaphoreType.DMA((n,)),
        ],
        compiler_params=pltpu.CompilerParams(collective_id=0),
    )(x, Wq, Wk, Wv, Wo, jnp.asarray(cosp), jnp.asarray(sinp), jnp.asarray(Pp))


# device time: 25740 ns/iter; 1.7051x vs baseline; 1.0319x over previous
import os

import numpy as np
import jax
import jax.numpy as jnp
from jax import lax
from jax.experimental import pallas as pl
from jax.experimental.pallas import tpu as pltpu

N_DEV = 8
B, SQ, D = 2, 256, 768
HL, DH = 4, 64
HD = HL * DH
SEG = SQ // N_DEV


def _consts():
    inv = 1.0 / (10000.0 ** (np.arange(0, DH, 2) / DH))
    pos = np.arange(SQ)[:, None] * inv[None, :]
    cos = np.repeat(np.cos(pos), 2, axis=-1)
    sin = np.repeat(np.sin(pos), 2, axis=-1)
    cosp = np.tile(cos, (1, HL)).astype(np.float32)
    sinp = np.tile(sin, (1, HL)).astype(np.float32)
    P = np.zeros((DH, DH), np.float32)
    for k in range(DH // 2):
        P[2 * k + 1, 2 * k] = -1.0
        P[2 * k, 2 * k + 1] = 1.0
    Pp = np.kron(np.eye(HL, dtype=np.float32), P)
    return cosp, sinp, Pp


def _body(x_ref, wbig_ref, wo_ref, cos_ref, sin_ref,
          out_ref, q_sc, rs_send, rs_recv, ag_send, ag_recv,
          rs_ssem, rs_rsem, ag_ssem, ag_rsem):
    my = lax.axis_index("i")
    bf = jnp.bfloat16
    f32 = jnp.float32
    compute_only = os.environ.get("KERNEL_COMPUTE_ONLY") == "1"

    barrier_sem = pltpu.get_barrier_semaphore()
    for o in range(1, N_DEV):
        pl.semaphore_signal(barrier_sem, inc=1,
                            device_id=(lax.rem(my + o, N_DEV),),
                            device_id_type=pl.DeviceIdType.MESH)
    pl.semaphore_wait(barrier_sem, N_DEV - 1)

    cos = cos_ref[:, :]
    sin = sin_ref[:, :]
    wo = wo_ref[:, :]

    wbig = wbig_ref[:, :]
    colk = lax.broadcasted_iota(jnp.int32, (SQ, HD), 1)
    ks, vs = [], []
    for b in range(B):
        xb = x_ref[b, :, :].astype(bf)
        big = jnp.dot(xb, wbig, preferred_element_type=f32)
        q, k, v = big[:, :HD], big[:, HD:2 * HD], big[:, 2 * HD:3 * HD]
        qp, kp = big[:, 3 * HD:4 * HD], big[:, 4 * HD:]
        q_sc[b, :, :] = ((q * cos + qp * sin) * 0.125).astype(bf)
        kr = (k * cos + kp * sin).astype(bf)
        vb = v.astype(bf)
        kh, vh = [], []
        for h in range(HL):
            m = (colk >= h * DH) & (colk < (h + 1) * DH)
            kh.append(jnp.where(m, kr, 0))
            vh.append(jnp.where(m, vb, 0))
        ks.append(kh)
        vs.append(vh)

    def attn_block(r0, nrows):
        for b in range(B):
            qr = q_sc[b, pl.ds(r0, nrows), :]
            ctx = jnp.zeros((nrows, HD), f32)
            for h in range(HL):
                s = lax.dot_general(qr, ks[b][h], (((1,), (1,)), ((), ())),
                                    preferred_element_type=f32)
                w = jnp.exp(s.astype(bf))
                w = w * (jnp.bfloat16(1.0) / jnp.sum(w, axis=-1,
                                                     keepdims=True))
                ctx = ctx + jnp.dot(w, vs[b][h], preferred_element_type=f32)
            out_ref[b, pl.ds(r0, nrows), :] = jnp.dot(
                ctx.astype(bf), wo, preferred_element_type=f32)

    def send_to(p):
        o = lax.rem(p - my + N_DEV, N_DEV)
        off = pl.multiple_of((o - 1) * SEG, SEG)
        src_lo = pl.multiple_of(p * SEG, SEG)
        rs_send[:, pl.ds(off, SEG), :] = (
            out_ref[:, pl.ds(src_lo, SEG), :].astype(bf))
        pltpu.make_async_remote_copy(
            src_ref=rs_send.at[:, pl.ds(off, SEG), :],
            dst_ref=rs_recv.at[:, pl.ds(off, SEG), :],
            send_sem=rs_ssem.at[o - 1],
            recv_sem=rs_rsem.at[o - 1],
            device_id=(p,),
            device_id_type=pl.DeviceIdType.MESH,
        ).start()

    if compute_only:
        nb = int(os.environ.get("KERNEL_BLOCK_ROWS", "64"))
        for q in range(SQ // nb):
            attn_block(pl.multiple_of(jnp.int32(q) * nb, nb), nb)
        return

    other_half = 1 - my // 4
    attn_block(pl.multiple_of(other_half * (SQ // 2), SQ // 2), SQ // 2)
    for j in range(4):
        send_to(4 * other_half + j)
    q_sib = jnp.bitwise_xor(my // 2, 1)
    attn_block(pl.multiple_of(q_sib * (2 * SEG), 2 * SEG), 2 * SEG)
    send_to(2 * q_sib)
    send_to(2 * q_sib + 1)
    partner = jnp.bitwise_xor(my, 1)
    attn_block(pl.multiple_of(partner * SEG, SEG), SEG)
    send_to(partner)
    attn_block(pl.multiple_of(my * SEG, SEG), SEG)

    my_lo = pl.multiple_of(my * SEG, SEG)
    acc = out_ref[:, pl.ds(my_lo, SEG), :]
    for o in (2, 6, 3, 5, 4, 1, 7):
        pltpu.make_async_remote_copy(
            src_ref=rs_send.at[:, pl.ds((o - 1) * SEG, SEG), :],
            dst_ref=rs_recv.at[:, pl.ds((o - 1) * SEG, SEG), :],
            send_sem=rs_ssem.at[o - 1],
            recv_sem=rs_rsem.at[o - 1],
            device_id=(my,),
            device_id_type=pl.DeviceIdType.MESH,
        ).wait_recv()
        acc = acc + rs_recv[:, (o - 1) * SEG:o * SEG, :].astype(f32)
    out_ref[:, pl.ds(my_lo, SEG), :] = acc

    ag_send[:, :, :] = acc.astype(bf)
    for o in (4, 3, 5, 2, 6, 1, 7):
        p = lax.rem(my + o, N_DEV)
        pltpu.make_async_remote_copy(
            src_ref=ag_send,
            dst_ref=ag_recv.at[:, pl.ds((o - 1) * SEG, SEG), :],
            send_sem=ag_ssem.at[o - 1],
            recv_sem=ag_rsem.at[o - 1],
            device_id=(p,),
            device_id_type=pl.DeviceIdType.MESH,
        ).start()
    for o in (1, 7, 2, 6, 3, 5, 4):
        rdma = pltpu.make_async_remote_copy(
            src_ref=ag_send,
            dst_ref=ag_recv.at[:, pl.ds((o - 1) * SEG, SEG), :],
            send_sem=ag_ssem.at[o - 1],
            recv_sem=ag_rsem.at[o - 1],
            device_id=(my,),
            device_id_type=pl.DeviceIdType.MESH,
        )
        rdma.wait_recv()
        s_o = pl.multiple_of(lax.rem(my - o + N_DEV, N_DEV) * SEG, SEG)
        out_ref[:, pl.ds(s_o, SEG), :] = (
            ag_recv[:, (o - 1) * SEG:o * SEG, :].astype(f32))
    for o in range(1, N_DEV):
        pltpu.make_async_remote_copy(
            src_ref=rs_send.at[:, pl.ds((o - 1) * SEG, SEG), :],
            dst_ref=rs_recv.at[:, pl.ds((o - 1) * SEG, SEG), :],
            send_sem=rs_ssem.at[o - 1],
            recv_sem=rs_rsem.at[o - 1],
            device_id=(my,),
            device_id_type=pl.DeviceIdType.MESH,
        ).wait_send()
        pltpu.make_async_remote_copy(
            src_ref=ag_send,
            dst_ref=ag_recv.at[:, pl.ds((o - 1) * SEG, SEG), :],
            send_sem=ag_ssem.at[o - 1],
            recv_sem=ag_rsem.at[o - 1],
            device_id=(my,),
            device_id_type=pl.DeviceIdType.MESH,
        ).wait_send()


def kernel(x, Wq, Wk, Wv, Wo):
    cosp, sinp, Pp = _consts()
    n = N_DEV - 1
    P = jnp.asarray(Pp)
    wbig = jnp.concatenate(
        [Wq, Wk, Wv, Wq @ P, Wk @ P], axis=1).astype(jnp.bfloat16)
    return pl.pallas_call(
        _body,
        out_shape=jax.ShapeDtypeStruct((B, SQ, D), jnp.float32),
        in_specs=[pl.BlockSpec(memory_space=pltpu.VMEM)] * 5,
        out_specs=pl.BlockSpec(memory_space=pltpu.VMEM),
        scratch_shapes=[
            pltpu.VMEM((B, SQ, HD), jnp.bfloat16),
            pltpu.VMEM((B, n * SEG, D), jnp.bfloat16),
            pltpu.VMEM((B, n * SEG, D), jnp.bfloat16),
            pltpu.VMEM((B, SEG, D), jnp.bfloat16),
            pltpu.VMEM((B, n * SEG, D), jnp.bfloat16),
            pltpu.SemaphoreType.DMA((n,)),
            pltpu.SemaphoreType.DMA((n,)),
            pltpu.SemaphoreType.DMA((n,)),
            pltpu.SemaphoreType.DMA((n,)),
        ],
        compiler_params=pltpu.CompilerParams(collective_id=0),
    )(x, wbig, Wo.astype(jnp.bfloat16),
      jnp.asarray(cosp), jnp.asarray(sinp))
